# baseline (speedup 1.0000x reference)
import functools

import jax
import jax.numpy as jnp
from jax import lax
from jax.experimental import pallas as pl
from jax.experimental.pallas import tpu as pltpu
from jax.experimental.pallas import tpu_sc as plsc

_NUM_CORES = 2
_NUM_SUBCORES = 16
_LANES = 16


def _unpack_lo(v):
    return lax.bitcast_convert_type(v << 16, jnp.float32)


def _unpack_hi(v):
    return lax.bitcast_convert_type(v & jnp.int32(-65536), jnp.float32)


@functools.lru_cache(maxsize=None)
def _make_sc_gather(n_nodes, L, H):
    NW = _NUM_CORES * _NUM_SUBCORES
    M = n_nodes // NW
    PB = 16
    NB = M // PB
    assert n_nodes == NW * NB * PB and NB % 2 == 0
    inv3L = 1.0 / (3.0 * L)
    inv3 = 1.0 / 3.0
    H2 = H // 2

    mesh = plsc.VectorSubcoreMesh(core_axis_name="c", subcore_axis_name="s")

    @functools.partial(
        pl.kernel,
        mesh=mesh,
        out_type=jax.ShapeDtypeStruct((n_nodes, H2), jnp.int32),
        scratch_types=[
            pltpu.VMEM((NB, PB * L), jnp.int32),
            pltpu.VMEM((NB, PB), jnp.int32),
            pltpu.VMEM((2, PB * L, H2), jnp.int32),
            pltpu.VMEM((2, PB, H2), jnp.int32),
            pltpu.VMEM((2, PB, H2), jnp.int32),
            pltpu.SemaphoreType.DMA,
            pltpu.SemaphoreType.DMA,
            pltpu.SemaphoreType.DMA,
            pltpu.SemaphoreType.DMA,
            pltpu.SemaphoreType.DMA,
            pltpu.SemaphoreType.DMA,
        ],
    )
    def sc_gather(xl_hbm, dg_hbm, ltab_hbm, dtab_hbm, out_hbm,
                  idxl_v, idxd_v, rowsl_v, rowsd_v, acc_v,
                  sem_l0, sem_l1, sem_d0, sem_d1, sem_o0, sem_o1):
        wid = lax.axis_index("s") * _NUM_CORES + lax.axis_index("c")
        pltpu.sync_copy(xl_hbm.at[wid], idxl_v)
        pltpu.sync_copy(dg_hbm.at[wid], idxd_v)
        sem_l = (sem_l0, sem_l1)
        sem_d = (sem_d0, sem_d1)
        sem_o = (sem_o0, sem_o1)

        def issue(b, s):
            pltpu.async_copy(ltab_hbm.at[idxl_v.at[b]], rowsl_v.at[s], sem_l[s])
            pltpu.async_copy(dtab_hbm.at[idxd_v.at[b]], rowsd_v.at[s], sem_d[s])

        def wait_gather(s):
            pltpu.make_async_copy(ltab_hbm.at[idxl_v.at[0]], rowsl_v.at[s],
                                  sem_l[s]).wait()
            pltpu.make_async_copy(dtab_hbm.at[idxd_v.at[0]], rowsd_v.at[s],
                                  sem_d[s]).wait()

        def compute(b, s):
            def node_body(j, carry2):
                r0 = j * L
                for c in range(H2 // _LANES):
                    sl = pl.ds(c * _LANES, _LANES)
                    v = rowsl_v[s, r0, sl]
                    lo = _unpack_lo(v)
                    hi = _unpack_hi(v)
                    for l in range(1, L):
                        v = rowsl_v[s, r0 + l, sl]
                        lo = lo + _unpack_lo(v)
                        hi = hi + _unpack_hi(v)
                    d = rowsd_v[s, j, sl]
                    rlo = lo * inv3L + _unpack_lo(d) * inv3
                    rhi = hi * inv3L + _unpack_hi(d) * inv3
                    ilo = lax.shift_right_logical(
                        lax.bitcast_convert_type(rlo, jnp.int32), 16)
                    ihi = (lax.bitcast_convert_type(rhi, jnp.int32)
                           & jnp.int32(-65536))
                    acc_v[s, j, sl] = ilo | ihi
                return carry2

            lax.fori_loop(0, PB, node_body, 0)
            pltpu.async_copy(acc_v.at[s],
                             out_hbm.at[pl.ds(wid * M + b * PB, PB)], sem_o[s])

        def wait_out(s):
            pltpu.make_async_copy(acc_v.at[s], out_hbm.at[pl.ds(0, PB)],
                                  sem_o[s]).wait()

        issue(0, 0)

        def pair_body(i, carry):
            b0 = i * 2
            issue(b0 + 1, 1)
            wait_gather(0)

            @pl.when(i > 0)
            def _():
                wait_out(0)

            compute(b0, 0)

            @pl.when(i < NB // 2 - 1)
            def _():
                issue(b0 + 2, 0)

            wait_gather(1)

            @pl.when(i > 0)
            def _():
                wait_out(1)

            compute(b0 + 1, 1)
            return carry

        lax.fori_loop(0, NB // 2, pair_body, 0)
        wait_out(0)
        wait_out(1)

    return sc_gather, NW, PB, NB


@functools.lru_cache(maxsize=None)
def _make_tc_combine(B, N, D, H):
    def body(x_ref, g_ref, w_ref, b_ref, t_ref, o_ref):
        xr = jnp.dot(x_ref[0], w_ref[...], preferred_element_type=jnp.float32)
        comb = g_ref[0].astype(jnp.float32) + (xr + b_ref[...]) * (1.0 / 3.0)
        o_ref[0] = jnp.concatenate([t_ref[...], comb], axis=0)

    return pl.pallas_call(
        body,
        grid=(B,),
        in_specs=[
            pl.BlockSpec((1, N, D), lambda g: (g, 0, 0)),
            pl.BlockSpec((1, N, H), lambda g: (g, 0, 0)),
            pl.BlockSpec((D, H), lambda g: (0, 0)),
            pl.BlockSpec((1, H), lambda g: (0, 0)),
            pl.BlockSpec((1, H), lambda g: (0, 0)),
        ],
        out_specs=pl.BlockSpec((1, N + 1, H), lambda g: (g, 0, 0)),
        out_shape=jax.ShapeDtypeStruct((B, N + 1, H), jnp.float32),
        compiler_params=pltpu.CompilerParams(
            dimension_semantics=("arbitrary",),
        ),
    )


def kernel(x_long, x_real, degree, long_table, real_W, real_b,
           degree_table, graph_token):
    B, N, L = x_long.shape
    D = x_real.shape[-1]
    H = long_table.shape[1]
    n_nodes = B * N

    def pack_bf16(tab):
        bf = tab.astype(jnp.bfloat16).reshape(tab.shape[0], H // 2, 2)
        return lax.bitcast_convert_type(bf, jnp.int32)

    sc_gather, NW, PB, NB = _make_sc_gather(n_nodes, L, H)
    xl_idx = x_long.astype(jnp.int32).reshape(NW, NB, PB * L)
    dg_idx = degree.astype(jnp.int32).reshape(NW, NB, PB)
    gath_i32 = sc_gather(xl_idx, dg_idx,
                         pack_bf16(long_table), pack_bf16(degree_table))
    gath = lax.bitcast_convert_type(gath_i32, jnp.bfloat16).reshape(
        n_nodes, H)

    tc = _make_tc_combine(B, N, D, H)
    return tc(x_real, gath.reshape(B, N, H), real_W,
              real_b.reshape(1, H), graph_token.reshape(1, H))


# speedup vs baseline: 1.6329x; 1.6329x over previous
import functools

import jax
import jax.numpy as jnp
from jax import lax
from jax.experimental import pallas as pl
from jax.experimental.pallas import tpu as pltpu
from jax.experimental.pallas import tpu_sc as plsc

_NUM_CORES = 2
_NUM_SUBCORES = 16
_LANES = 16


def _unpack_lo(v):
    return lax.bitcast_convert_type(v << 16, jnp.float32)


def _unpack_hi(v):
    return lax.bitcast_convert_type(v & jnp.int32(-65536), jnp.float32)


@functools.lru_cache(maxsize=None)
def _make_sc_gather(n_nodes, L, H):
    NW = _NUM_CORES * _NUM_SUBCORES
    M = n_nodes // NW
    PB = 16
    NB = M // PB
    assert n_nodes == NW * NB * PB and NB % 2 == 0
    inv3L = 1.0 / (3.0 * L)
    inv3 = 1.0 / 3.0
    H2 = H // 2

    mesh = plsc.VectorSubcoreMesh(core_axis_name="c", subcore_axis_name="s")

    @functools.partial(
        pl.kernel,
        mesh=mesh,
        out_type=jax.ShapeDtypeStruct((n_nodes, H2), jnp.int32),
        scratch_types=[
            pltpu.VMEM((NB, PB * L), jnp.int32),
            pltpu.VMEM((NB, PB), jnp.int32),
            pltpu.VMEM((2, PB * L, H2), jnp.int32),
            pltpu.VMEM((2, PB, H2), jnp.int32),
            pltpu.VMEM((2, PB, H2), jnp.int32),
            pltpu.SemaphoreType.DMA,
            pltpu.SemaphoreType.DMA,
            pltpu.SemaphoreType.DMA,
            pltpu.SemaphoreType.DMA,
            pltpu.SemaphoreType.DMA,
            pltpu.SemaphoreType.DMA,
        ],
    )
    def sc_gather(xl_hbm, dg_hbm, ltab_hbm, dtab_hbm, out_hbm,
                  idxl_v, idxd_v, rowsl_v, rowsd_v, acc_v,
                  sem_l0, sem_l1, sem_d0, sem_d1, sem_o0, sem_o1):
        wid = lax.axis_index("s") * _NUM_CORES + lax.axis_index("c")
        pltpu.sync_copy(xl_hbm.at[wid], idxl_v)
        pltpu.sync_copy(dg_hbm.at[wid], idxd_v)
        sem_l = (sem_l0, sem_l1)
        sem_d = (sem_d0, sem_d1)
        sem_o = (sem_o0, sem_o1)

        def issue(b, s):
            pltpu.async_copy(ltab_hbm.at[idxl_v.at[b]], rowsl_v.at[s], sem_l[s])
            pltpu.async_copy(dtab_hbm.at[idxd_v.at[b]], rowsd_v.at[s], sem_d[s])

        def wait_gather(s):
            pltpu.make_async_copy(ltab_hbm.at[idxl_v.at[0]], rowsl_v.at[s],
                                  sem_l[s]).wait()
            pltpu.make_async_copy(dtab_hbm.at[idxd_v.at[0]], rowsd_v.at[s],
                                  sem_d[s]).wait()

        def compute(b, s):
            def node_body(j, carry2):
                r0 = j * L
                for c in range(H2 // _LANES):
                    sl = pl.ds(c * _LANES, _LANES)
                    v = rowsl_v[s, r0, sl]
                    lo = _unpack_lo(v)
                    hi = _unpack_hi(v)
                    for l in range(1, L):
                        v = rowsl_v[s, r0 + l, sl]
                        lo = lo + _unpack_lo(v)
                        hi = hi + _unpack_hi(v)
                    d = rowsd_v[s, j, sl]
                    rlo = lo * inv3L + _unpack_lo(d) * inv3
                    rhi = hi * inv3L + _unpack_hi(d) * inv3
                    ilo = lax.shift_right_logical(
                        lax.bitcast_convert_type(rlo, jnp.int32), 16)
                    ihi = (lax.bitcast_convert_type(rhi, jnp.int32)
                           & jnp.int32(-65536))
                    acc_v[s, j, sl] = ilo | ihi
                return carry2

            lax.fori_loop(0, PB, node_body, 0)
            pltpu.async_copy(acc_v.at[s],
                             out_hbm.at[pl.ds(wid * M + b * PB, PB)], sem_o[s])

        def wait_out(s):
            pltpu.make_async_copy(acc_v.at[s], out_hbm.at[pl.ds(0, PB)],
                                  sem_o[s]).wait()

        issue(0, 0)

        def pair_body(i, carry):
            b0 = i * 2
            issue(b0 + 1, 1)
            wait_gather(0)

            @pl.when(i > 0)
            def _():
                wait_out(0)

            compute(b0, 0)

            @pl.when(i < NB // 2 - 1)
            def _():
                issue(b0 + 2, 0)

            wait_gather(1)

            @pl.when(i > 0)
            def _():
                wait_out(1)

            compute(b0 + 1, 1)
            return carry

        lax.fori_loop(0, NB // 2, pair_body, 0)
        wait_out(0)
        wait_out(1)

    return sc_gather, NW, PB, NB


@functools.lru_cache(maxsize=None)
def _make_tc_combine(B, N, D, H):
    H2 = H // 2

    def body(x_ref, g_ref, w_ref, b_ref, t_ref, o_ref):
        xr = jnp.dot(x_ref[0], w_ref[...], preferred_element_type=jnp.float32)
        v = g_ref[0]
        glo = lax.bitcast_convert_type(v << 16, jnp.float32)
        ghi = lax.bitcast_convert_type(v & jnp.int32(-65536), jnp.float32)
        gf = jnp.concatenate([glo, ghi], axis=-1)
        comb = gf + (xr + b_ref[...]) * (1.0 / 3.0)
        o_ref[0] = jnp.concatenate([t_ref[...], comb], axis=0)

    return pl.pallas_call(
        body,
        grid=(B,),
        in_specs=[
            pl.BlockSpec((1, N, D), lambda g: (g, 0, 0)),
            pl.BlockSpec((1, N, H2), lambda g: (g, 0, 0)),
            pl.BlockSpec((D, H), lambda g: (0, 0)),
            pl.BlockSpec((1, H), lambda g: (0, 0)),
            pl.BlockSpec((1, H), lambda g: (0, 0)),
        ],
        out_specs=pl.BlockSpec((1, N + 1, H), lambda g: (g, 0, 0)),
        out_shape=jax.ShapeDtypeStruct((B, N + 1, H), jnp.float32),
        compiler_params=pltpu.CompilerParams(
            dimension_semantics=("arbitrary",),
        ),
    )


def kernel(x_long, x_real, degree, long_table, real_W, real_b,
           degree_table, graph_token):
    B, N, L = x_long.shape
    D = x_real.shape[-1]
    H = long_table.shape[1]
    H2 = H // 2
    n_nodes = B * N

    def pack_bf16(tab):
        bf = tab.astype(jnp.bfloat16)
        lo = lax.bitcast_convert_type(bf[:, :H2], jnp.uint16).astype(
            jnp.uint32)
        hi = lax.bitcast_convert_type(bf[:, H2:], jnp.uint16).astype(
            jnp.uint32)
        return lax.bitcast_convert_type(lo | (hi << 16), jnp.int32)

    sc_gather, NW, PB, NB = _make_sc_gather(n_nodes, L, H)
    xl_idx = x_long.astype(jnp.int32).reshape(NW, NB, PB * L)
    dg_idx = degree.astype(jnp.int32).reshape(NW, NB, PB)
    gath_i32 = sc_gather(xl_idx, dg_idx,
                         pack_bf16(long_table), pack_bf16(degree_table))

    tc = _make_tc_combine(B, N, D, H)
    return tc(x_real, gath_i32.reshape(B, N, H2), real_W,
              real_b.reshape(1, H), graph_token.reshape(1, H))


# speedup vs baseline: 1.6331x; 1.0001x over previous
import functools

import jax
import jax.numpy as jnp
from jax import lax
from jax.experimental import pallas as pl
from jax.experimental.pallas import tpu as pltpu
from jax.experimental.pallas import tpu_sc as plsc

_NUM_CORES = 2
_NUM_SUBCORES = 16
_LANES = 16


def _unpack_lo(v):
    return lax.bitcast_convert_type(v << 16, jnp.float32)


def _unpack_hi(v):
    return lax.bitcast_convert_type(v & jnp.int32(-65536), jnp.float32)


@functools.lru_cache(maxsize=None)
def _make_sc_gather(n_nodes, L, H):
    NW = _NUM_CORES * _NUM_SUBCORES
    M = n_nodes // NW
    PB = 16
    NB = M // PB
    assert n_nodes == NW * NB * PB and NB % 2 == 0
    inv3L = 1.0 / (3.0 * L)
    inv3 = 1.0 / 3.0
    H2 = H // 2

    mesh = plsc.VectorSubcoreMesh(core_axis_name="c", subcore_axis_name="s")

    @functools.partial(
        pl.kernel,
        mesh=mesh,
        out_type=jax.ShapeDtypeStruct((n_nodes, H2), jnp.int32),
        scratch_types=[
            pltpu.VMEM((NB, PB * L), jnp.int32),
            pltpu.VMEM((NB, PB), jnp.int32),
            pltpu.VMEM((2, PB * L, H2), jnp.int32),
            pltpu.VMEM((2, PB, H2), jnp.int32),
            pltpu.VMEM((2, PB, H2), jnp.int32),
            pltpu.SemaphoreType.DMA,
            pltpu.SemaphoreType.DMA,
            pltpu.SemaphoreType.DMA,
            pltpu.SemaphoreType.DMA,
            pltpu.SemaphoreType.DMA,
            pltpu.SemaphoreType.DMA,
        ],
    )
    def sc_gather(xl_hbm, dg_hbm, ltab_hbm, dtab_hbm, out_hbm,
                  idxl_v, idxd_v, rowsl_v, rowsd_v, acc_v,
                  sem_l0, sem_l1, sem_d0, sem_d1, sem_o0, sem_o1):
        wid = lax.axis_index("s") * _NUM_CORES + lax.axis_index("c")
        pltpu.sync_copy(xl_hbm.at[wid], idxl_v)
        pltpu.sync_copy(dg_hbm.at[wid], idxd_v)
        sem_l = (sem_l0, sem_l1)
        sem_d = (sem_d0, sem_d1)
        sem_o = (sem_o0, sem_o1)

        def issue(b, s):
            pltpu.async_copy(ltab_hbm.at[idxl_v.at[b]], rowsl_v.at[s], sem_l[s])
            pltpu.async_copy(dtab_hbm.at[idxd_v.at[b]], rowsd_v.at[s], sem_d[s])

        def wait_gather(s):
            pltpu.make_async_copy(ltab_hbm.at[idxl_v.at[0]], rowsl_v.at[s],
                                  sem_l[s]).wait()
            pltpu.make_async_copy(dtab_hbm.at[idxd_v.at[0]], rowsd_v.at[s],
                                  sem_d[s]).wait()

        def compute(b, s):
            def node_body(j, carry2):
                r0 = j * L
                for c in range(H2 // _LANES):
                    sl = pl.ds(c * _LANES, _LANES)
                    v = rowsl_v[s, r0, sl]
                    lo = _unpack_lo(v)
                    hi = _unpack_hi(v)
                    for l in range(1, L):
                        v = rowsl_v[s, r0 + l, sl]
                        lo = lo + _unpack_lo(v)
                        hi = hi + _unpack_hi(v)
                    d = rowsd_v[s, j, sl]
                    rlo = lo * inv3L + _unpack_lo(d) * inv3
                    rhi = hi * inv3L + _unpack_hi(d) * inv3
                    ilo = lax.shift_right_logical(
                        lax.bitcast_convert_type(rlo, jnp.int32), 16)
                    ihi = (lax.bitcast_convert_type(rhi, jnp.int32)
                           & jnp.int32(-65536))
                    acc_v[s, j, sl] = ilo | ihi
                return carry2

            lax.fori_loop(0, PB, node_body, 0)
            pltpu.async_copy(acc_v.at[s],
                             out_hbm.at[pl.ds(wid * M + b * PB, PB)], sem_o[s])

        def wait_out(s):
            pltpu.make_async_copy(acc_v.at[s], out_hbm.at[pl.ds(0, PB)],
                                  sem_o[s]).wait()

        issue(0, 0)

        def pair_body(i, carry):
            b0 = i * 2
            issue(b0 + 1, 1)
            wait_gather(0)

            @pl.when(i > 0)
            def _():
                wait_out(0)

            compute(b0, 0)

            @pl.when(i < NB // 2 - 1)
            def _():
                issue(b0 + 2, 0)

            wait_gather(1)

            @pl.when(i > 0)
            def _():
                wait_out(1)

            compute(b0 + 1, 1)
            return carry

        lax.fori_loop(0, NB // 2, pair_body, 0)
        wait_out(0)
        wait_out(1)

    return sc_gather, NW, PB, NB


@functools.lru_cache(maxsize=None)
def _make_tc_combine(B, N, D, H):
    H2 = H // 2

    def body(x_ref, g_ref, w_ref, b_ref, t_ref, o_ref):
        xr = jnp.dot(x_ref[0].astype(jnp.bfloat16), w_ref[...],
                     preferred_element_type=jnp.float32)
        v = g_ref[0]
        glo = lax.bitcast_convert_type(v << 16, jnp.float32)
        ghi = lax.bitcast_convert_type(v & jnp.int32(-65536), jnp.float32)
        gf = jnp.concatenate([glo, ghi], axis=-1)
        comb = gf + (xr + b_ref[...]) * (1.0 / 3.0)
        o_ref[0] = jnp.concatenate([t_ref[...], comb], axis=0)

    return pl.pallas_call(
        body,
        grid=(B,),
        in_specs=[
            pl.BlockSpec((1, N, D), lambda g: (g, 0, 0)),
            pl.BlockSpec((1, N, H2), lambda g: (g, 0, 0)),
            pl.BlockSpec((D, H), lambda g: (0, 0)),
            pl.BlockSpec((1, H), lambda g: (0, 0)),
            pl.BlockSpec((1, H), lambda g: (0, 0)),
        ],
        out_specs=pl.BlockSpec((1, N + 1, H), lambda g: (g, 0, 0)),
        out_shape=jax.ShapeDtypeStruct((B, N + 1, H), jnp.float32),
        compiler_params=pltpu.CompilerParams(
            dimension_semantics=("arbitrary",),
        ),
    )


def kernel(x_long, x_real, degree, long_table, real_W, real_b,
           degree_table, graph_token):
    B, N, L = x_long.shape
    D = x_real.shape[-1]
    H = long_table.shape[1]
    H2 = H // 2
    n_nodes = B * N

    def pack_bf16(tab):
        bf = tab.astype(jnp.bfloat16)
        lo = lax.bitcast_convert_type(bf[:, :H2], jnp.uint16).astype(
            jnp.uint32)
        hi = lax.bitcast_convert_type(bf[:, H2:], jnp.uint16).astype(
            jnp.uint32)
        return lax.bitcast_convert_type(lo | (hi << 16), jnp.int32)

    sc_gather, NW, PB, NB = _make_sc_gather(n_nodes, L, H)
    xl_idx = x_long.astype(jnp.int32).reshape(NW, NB, PB * L)
    dg_idx = degree.astype(jnp.int32).reshape(NW, NB, PB)
    gath_i32 = sc_gather(xl_idx, dg_idx,
                         pack_bf16(long_table), pack_bf16(degree_table))

    tc = _make_tc_combine(B, N, D, H)
    return tc(x_real, gath_i32.reshape(B, N, H2),
              real_W.astype(jnp.bfloat16),
              real_b.reshape(1, H), graph_token.reshape(1, H))


# speedup vs baseline: 1.6500x; 1.0103x over previous
import functools

import jax
import jax.numpy as jnp
from jax import lax
from jax.experimental import pallas as pl
from jax.experimental.pallas import tpu as pltpu
from jax.experimental.pallas import tpu_sc as plsc

_NUM_CORES = 2
_NUM_SUBCORES = 16
_LANES = 16


def _unpack_lo(v):
    return lax.bitcast_convert_type(v << 16, jnp.float32)


def _unpack_hi(v):
    return lax.bitcast_convert_type(v & jnp.int32(-65536), jnp.float32)


@functools.lru_cache(maxsize=None)
def _make_sc_gather(n_nodes, L, H):
    NW = _NUM_CORES * _NUM_SUBCORES
    M = n_nodes // NW
    PB = 16
    NB = M // PB
    assert n_nodes == NW * NB * PB and NB % 2 == 0
    inv3L = 1.0 / (3.0 * L)
    inv3 = 1.0 / 3.0
    H2 = H // 2

    mesh = plsc.VectorSubcoreMesh(core_axis_name="c", subcore_axis_name="s")

    @functools.partial(
        pl.kernel,
        mesh=mesh,
        out_type=jax.ShapeDtypeStruct((n_nodes, H2), jnp.int32),
        scratch_types=[
            pltpu.VMEM((NB * PB * L,), jnp.int32),
            pltpu.VMEM((NB * PB,), jnp.int32),
            pltpu.VMEM((2, PB * L, H2), jnp.int32),
            pltpu.VMEM((2, PB, H2), jnp.int32),
            pltpu.VMEM((2, PB, H2), jnp.int32),
            pltpu.SemaphoreType.DMA,
            pltpu.SemaphoreType.DMA,
            pltpu.SemaphoreType.DMA,
            pltpu.SemaphoreType.DMA,
            pltpu.SemaphoreType.DMA,
            pltpu.SemaphoreType.DMA,
        ],
    )
    def sc_gather(xl_hbm, dg_hbm, ltab_hbm, dtab_hbm, out_hbm,
                  idxl_v, idxd_v, rowsl_v, rowsd_v, acc_v,
                  sem_l0, sem_l1, sem_d0, sem_d1, sem_o0, sem_o1):
        wid = lax.axis_index("s") * _NUM_CORES + lax.axis_index("c")
        pltpu.sync_copy(xl_hbm.at[pl.ds(wid * (M * L), M * L)], idxl_v)
        pltpu.sync_copy(dg_hbm.at[pl.ds(wid * M, M)], idxd_v)
        sem_l = (sem_l0, sem_l1)
        sem_d = (sem_d0, sem_d1)
        sem_o = (sem_o0, sem_o1)

        def issue(b, s):
            pltpu.async_copy(ltab_hbm.at[idxl_v.at[pl.ds(b * PB * L, PB * L)]],
                             rowsl_v.at[s], sem_l[s])
            pltpu.async_copy(dtab_hbm.at[idxd_v.at[pl.ds(b * PB, PB)]],
                             rowsd_v.at[s], sem_d[s])

        def wait_gather(s):
            pltpu.make_async_copy(ltab_hbm.at[idxl_v.at[pl.ds(0, PB * L)]],
                                  rowsl_v.at[s], sem_l[s]).wait()
            pltpu.make_async_copy(dtab_hbm.at[idxd_v.at[pl.ds(0, PB)]],
                                  rowsd_v.at[s], sem_d[s]).wait()

        def compute(b, s):
            def node_body(j, carry2):
                r0 = j * L
                for c in range(H2 // _LANES):
                    sl = pl.ds(c * _LANES, _LANES)
                    v = rowsl_v[s, r0, sl]
                    lo = _unpack_lo(v)
                    hi = _unpack_hi(v)
                    for l in range(1, L):
                        v = rowsl_v[s, r0 + l, sl]
                        lo = lo + _unpack_lo(v)
                        hi = hi + _unpack_hi(v)
                    d = rowsd_v[s, j, sl]
                    rlo = lo * inv3L + _unpack_lo(d) * inv3
                    rhi = hi * inv3L + _unpack_hi(d) * inv3
                    ilo = lax.shift_right_logical(
                        lax.bitcast_convert_type(rlo, jnp.int32), 16)
                    ihi = (lax.bitcast_convert_type(rhi, jnp.int32)
                           & jnp.int32(-65536))
                    acc_v[s, j, sl] = ilo | ihi
                return carry2

            lax.fori_loop(0, PB, node_body, 0)
            pltpu.async_copy(acc_v.at[s],
                             out_hbm.at[pl.ds(wid * M + b * PB, PB)], sem_o[s])

        def wait_out(s):
            pltpu.make_async_copy(acc_v.at[s], out_hbm.at[pl.ds(0, PB)],
                                  sem_o[s]).wait()

        issue(0, 0)

        def pair_body(i, carry):
            b0 = i * 2
            issue(b0 + 1, 1)
            wait_gather(0)

            @pl.when(i > 0)
            def _():
                wait_out(0)

            compute(b0, 0)

            @pl.when(i < NB // 2 - 1)
            def _():
                issue(b0 + 2, 0)

            wait_gather(1)

            @pl.when(i > 0)
            def _():
                wait_out(1)

            compute(b0 + 1, 1)
            return carry

        lax.fori_loop(0, NB // 2, pair_body, 0)
        wait_out(0)
        wait_out(1)

    return sc_gather, NW, PB, NB


@functools.lru_cache(maxsize=None)
def _make_tc_combine(B, N, D, H):
    H2 = H // 2

    def body(x_ref, g_ref, w_ref, b_ref, t_ref, o_ref):
        xr = jnp.dot(x_ref[0].astype(jnp.bfloat16), w_ref[...],
                     preferred_element_type=jnp.float32)
        v = g_ref[0]
        glo = lax.bitcast_convert_type(v << 16, jnp.float32)
        ghi = lax.bitcast_convert_type(v & jnp.int32(-65536), jnp.float32)
        gf = jnp.concatenate([glo, ghi], axis=-1)
        comb = gf + (xr + b_ref[...]) * (1.0 / 3.0)
        o_ref[0] = jnp.concatenate([t_ref[...], comb], axis=0)

    return pl.pallas_call(
        body,
        grid=(B,),
        in_specs=[
            pl.BlockSpec((1, N, D), lambda g: (g, 0, 0)),
            pl.BlockSpec((1, N, H2), lambda g: (g, 0, 0)),
            pl.BlockSpec((D, H), lambda g: (0, 0)),
            pl.BlockSpec((1, H), lambda g: (0, 0)),
            pl.BlockSpec((1, H), lambda g: (0, 0)),
        ],
        out_specs=pl.BlockSpec((1, N + 1, H), lambda g: (g, 0, 0)),
        out_shape=jax.ShapeDtypeStruct((B, N + 1, H), jnp.float32),
        compiler_params=pltpu.CompilerParams(
            dimension_semantics=("parallel",),
        ),
    )


def kernel(x_long, x_real, degree, long_table, real_W, real_b,
           degree_table, graph_token):
    B, N, L = x_long.shape
    D = x_real.shape[-1]
    H = long_table.shape[1]
    H2 = H // 2
    n_nodes = B * N

    def pack_bf16(tab):
        bf = tab.astype(jnp.bfloat16)
        lo = lax.bitcast_convert_type(bf[:, :H2], jnp.uint16).astype(
            jnp.uint32)
        hi = lax.bitcast_convert_type(bf[:, H2:], jnp.uint16).astype(
            jnp.uint32)
        return lax.bitcast_convert_type(lo | (hi << 16), jnp.int32)

    sc_gather, NW, PB, NB = _make_sc_gather(n_nodes, L, H)
    xl_idx = x_long.astype(jnp.int32).reshape(n_nodes * L)
    dg_idx = degree.astype(jnp.int32).reshape(n_nodes)
    gath_i32 = sc_gather(xl_idx, dg_idx,
                         pack_bf16(long_table), pack_bf16(degree_table))

    tc = _make_tc_combine(B, N, D, H)
    return tc(x_real, gath_i32.reshape(B, N, H2),
              real_W.astype(jnp.bfloat16),
              real_b.reshape(1, H), graph_token.reshape(1, H))


# speedup vs baseline: 1.8318x; 1.1102x over previous
import functools

import jax
import jax.numpy as jnp
from jax import lax
from jax.experimental import pallas as pl
from jax.experimental.pallas import tpu as pltpu
from jax.experimental.pallas import tpu_sc as plsc

_NUM_CORES = 2
_NUM_SUBCORES = 16
_LANES = 16
_RING = 4


def _unpack_lo(v):
    return lax.bitcast_convert_type(v << 16, jnp.float32)


def _unpack_hi(v):
    return lax.bitcast_convert_type(v & jnp.int32(-65536), jnp.float32)


@functools.lru_cache(maxsize=None)
def _make_sc_gather(n_nodes, L, H):
    NW = _NUM_CORES * _NUM_SUBCORES
    M = n_nodes // NW
    PB = 8
    NB = M // PB
    LD = L + 1
    assert n_nodes == NW * NB * PB and NB % _RING == 0
    assert PB * LD <= 128
    assert (PB * LD) % 8 == 0
    inv3L = 1.0 / (3.0 * L)
    inv3 = 1.0 / 3.0
    H2 = H // 2

    mesh = plsc.VectorSubcoreMesh(core_axis_name="c", subcore_axis_name="s")

    @functools.partial(
        pl.kernel,
        mesh=mesh,
        out_type=jax.ShapeDtypeStruct((n_nodes, H2), jnp.int32),
        scratch_types=[
            pltpu.VMEM((NB * PB * LD,), jnp.int32),
            pltpu.VMEM((_RING, PB * LD, H2), jnp.int32),
            pltpu.VMEM((_RING, PB, H2), jnp.int32),
        ] + [pltpu.SemaphoreType.DMA] * (2 * _RING),
    )
    def sc_gather(idx_hbm, tab_hbm, out_hbm, idx_v, rows_v, acc_v, *sems):
        sem_g = sems[:_RING]
        sem_o = sems[_RING:]
        wid = lax.axis_index("s") * _NUM_CORES + lax.axis_index("c")
        pltpu.sync_copy(idx_hbm.at[pl.ds(wid * (M * LD), M * LD)], idx_v)

        def issue(b, s):
            pltpu.async_copy(
                tab_hbm.at[idx_v.at[pl.ds(b * PB * LD, PB * LD)]],
                rows_v.at[s], sem_g[s])

        def wait_gather(s):
            pltpu.make_async_copy(
                tab_hbm.at[idx_v.at[pl.ds(0, PB * LD)]],
                rows_v.at[s], sem_g[s]).wait()

        def compute(b, s):
            def node_body(j, carry2):
                r0 = j * LD
                for c in range(H2 // _LANES):
                    sl = pl.ds(c * _LANES, _LANES)
                    v = rows_v[s, r0, sl]
                    lo = _unpack_lo(v)
                    hi = _unpack_hi(v)
                    for l in range(1, L):
                        v = rows_v[s, r0 + l, sl]
                        lo = lo + _unpack_lo(v)
                        hi = hi + _unpack_hi(v)
                    d = rows_v[s, r0 + L, sl]
                    rlo = lo * inv3L + _unpack_lo(d) * inv3
                    rhi = hi * inv3L + _unpack_hi(d) * inv3
                    ilo = lax.shift_right_logical(
                        lax.bitcast_convert_type(rlo, jnp.int32), 16)
                    ihi = (lax.bitcast_convert_type(rhi, jnp.int32)
                           & jnp.int32(-65536))
                    acc_v[s, j, sl] = ilo | ihi
                return carry2

            lax.fori_loop(0, PB, node_body, 0)
            pltpu.async_copy(acc_v.at[s],
                             out_hbm.at[pl.ds(wid * M + b * PB, PB)], sem_o[s])

        def wait_out(s):
            pltpu.make_async_copy(acc_v.at[s], out_hbm.at[pl.ds(0, PB)],
                                  sem_o[s]).wait()

        for s in range(_RING):
            issue(s, s)

        def ring_body(i, carry):
            for s in range(_RING):
                b = i * _RING + s
                wait_gather(s)

                @pl.when(i > 0)
                def _():
                    wait_out(s)

                compute(b, s)

                @pl.when(b + _RING < NB)
                def _():
                    issue(b + _RING, s)
            return carry

        lax.fori_loop(0, NB // _RING, ring_body, 0)
        for s in range(_RING):
            wait_out(s)

    return sc_gather, NW, PB, NB, LD


@functools.lru_cache(maxsize=None)
def _make_tc_combine(B, N, D, H, G):
    H2 = H // 2

    def body(x_ref, g_ref, w_ref, b_ref, t_ref, o_ref):
        x2 = x_ref[...].reshape(G * N, D)
        xr = jnp.dot(x2, w_ref[...], preferred_element_type=jnp.float32)
        xr = ((xr + b_ref[...]) * (1.0 / 3.0)).reshape(G, N, H)
        v = g_ref[...]
        glo = lax.bitcast_convert_type(v << 16, jnp.float32)
        ghi = lax.bitcast_convert_type(v & jnp.int32(-65536), jnp.float32)
        gf = jnp.concatenate([glo, ghi], axis=-1)
        comb = gf + xr
        tok = jnp.broadcast_to(t_ref[...].reshape(1, 1, H), (G, 1, H))
        o_ref[...] = jnp.concatenate([tok, comb], axis=1)

    return pl.pallas_call(
        body,
        grid=(B // G,),
        in_specs=[
            pl.BlockSpec((G, N, D), lambda g: (g, 0, 0)),
            pl.BlockSpec((G, N, H2), lambda g: (g, 0, 0)),
            pl.BlockSpec((D, H), lambda g: (0, 0)),
            pl.BlockSpec((1, H), lambda g: (0, 0)),
            pl.BlockSpec((1, H), lambda g: (0, 0)),
        ],
        out_specs=pl.BlockSpec((G, N + 1, H), lambda g: (g, 0, 0)),
        out_shape=jax.ShapeDtypeStruct((B, N + 1, H), jnp.float32),
        compiler_params=pltpu.CompilerParams(
            dimension_semantics=("parallel",),
        ),
    )


def kernel(x_long, x_real, degree, long_table, real_W, real_b,
           degree_table, graph_token):
    B, N, L = x_long.shape
    D = x_real.shape[-1]
    H = long_table.shape[1]
    H2 = H // 2
    n_nodes = B * N
    n_long = long_table.shape[0]

    def pack_bf16(tab):
        bf = tab.astype(jnp.bfloat16)
        lo = lax.bitcast_convert_type(bf[:, :H2], jnp.uint16).astype(
            jnp.uint32)
        hi = lax.bitcast_convert_type(bf[:, H2:], jnp.uint16).astype(
            jnp.uint32)
        return lax.bitcast_convert_type(lo | (hi << 16), jnp.int32)

    sc_gather, NW, PB, NB, LD = _make_sc_gather(n_nodes, L, H)
    tab = jnp.concatenate([pack_bf16(long_table), pack_bf16(degree_table)])
    idx = jnp.concatenate(
        [x_long.astype(jnp.int32).reshape(n_nodes, L),
         degree.astype(jnp.int32).reshape(n_nodes, 1) + n_long],
        axis=1).reshape(n_nodes * LD)
    gath_i32 = sc_gather(idx, tab)

    tc = _make_tc_combine(B, N, D, H, 4)
    return tc(x_real.astype(jnp.bfloat16), gath_i32.reshape(B, N, H2),
              real_W.astype(jnp.bfloat16),
              real_b.reshape(1, H), graph_token.reshape(1, H))


# speedup vs baseline: 1.8754x; 1.0238x over previous
import functools

import jax
import jax.numpy as jnp
from jax import lax
from jax.experimental import pallas as pl
from jax.experimental.pallas import tpu as pltpu
from jax.experimental.pallas import tpu_sc as plsc

_NUM_CORES = 2
_NUM_SUBCORES = 16
_LANES = 16
_RING = 4


def _unpack_lo(v):
    return lax.bitcast_convert_type(v << 16, jnp.float32)


def _unpack_hi(v):
    return lax.bitcast_convert_type(v & jnp.int32(-65536), jnp.float32)


@functools.lru_cache(maxsize=None)
def _make_sc_gather(n_nodes, L, H):
    NW = _NUM_CORES * _NUM_SUBCORES
    M = n_nodes // NW
    PB = 8
    NB = M // PB
    LD = L + 1
    assert n_nodes == NW * NB * PB and NB % _RING == 0
    assert PB * LD <= 128
    assert (PB * LD) % 8 == 0
    inv3L = 1.0 / (3.0 * L)
    inv3 = 1.0 / 3.0
    H2 = H // 2

    mesh = plsc.VectorSubcoreMesh(core_axis_name="c", subcore_axis_name="s")

    @functools.partial(
        pl.kernel,
        mesh=mesh,
        out_type=jax.ShapeDtypeStruct((n_nodes, H2), jnp.int32),
        scratch_types=[
            pltpu.VMEM((NB * PB * LD,), jnp.int32),
            pltpu.VMEM((_RING, PB * LD, H2), jnp.int32),
            pltpu.VMEM((_RING, PB, H2), jnp.int32),
        ] + [pltpu.SemaphoreType.DMA] * (2 * _RING),
    )
    def sc_gather(idx_hbm, tab_hbm, out_hbm, idx_v, rows_v, acc_v, *sems):
        sem_g = sems[:_RING]
        sem_o = sems[_RING:]
        wid = lax.axis_index("s") * _NUM_CORES + lax.axis_index("c")
        pltpu.sync_copy(idx_hbm.at[pl.ds(wid * (M * LD), M * LD)], idx_v)

        def issue(b, s):
            pltpu.async_copy(
                tab_hbm.at[idx_v.at[pl.ds(b * PB * LD, PB * LD)]],
                rows_v.at[s], sem_g[s])

        def wait_gather(s):
            pltpu.make_async_copy(
                tab_hbm.at[idx_v.at[pl.ds(0, PB * LD)]],
                rows_v.at[s], sem_g[s]).wait()

        def compute(b, s):
            def node_body(j, carry2):
                r0 = j * LD
                for c in range(H2 // _LANES):
                    sl = pl.ds(c * _LANES, _LANES)
                    v = rows_v[s, r0, sl]
                    lo = _unpack_lo(v)
                    hi = _unpack_hi(v)
                    for l in range(1, L):
                        v = rows_v[s, r0 + l, sl]
                        lo = lo + _unpack_lo(v)
                        hi = hi + _unpack_hi(v)
                    d = rows_v[s, r0 + L, sl]
                    rlo = lo * inv3L + _unpack_lo(d) * inv3
                    rhi = hi * inv3L + _unpack_hi(d) * inv3
                    ilo = lax.shift_right_logical(
                        lax.bitcast_convert_type(rlo, jnp.int32), 16)
                    ihi = (lax.bitcast_convert_type(rhi, jnp.int32)
                           & jnp.int32(-65536))
                    acc_v[s, j, sl] = ilo | ihi
                return carry2

            lax.fori_loop(0, PB, node_body, 0)
            pltpu.async_copy(acc_v.at[s],
                             out_hbm.at[pl.ds(wid * M + b * PB, PB)], sem_o[s])

        def wait_out(s):
            pltpu.make_async_copy(acc_v.at[s], out_hbm.at[pl.ds(0, PB)],
                                  sem_o[s]).wait()

        for s in range(_RING):
            issue(s, s)

        def ring_body(i, carry):
            for s in range(_RING):
                b = i * _RING + s
                wait_gather(s)

                @pl.when(i > 0)
                def _():
                    wait_out(s)

                compute(b, s)

                @pl.when(b + _RING < NB)
                def _():
                    issue(b + _RING, s)
            return carry

        lax.fori_loop(0, NB // _RING, ring_body, 0)
        for s in range(_RING):
            wait_out(s)

    return sc_gather, NW, PB, NB, LD


@functools.lru_cache(maxsize=None)
def _make_tc_combine(B, N, D, H, G):
    H2 = H // 2

    def body(x_ref, g_ref, w_ref, b_ref, t_ref, o_ref):
        x2 = x_ref[...].reshape(G * N, D).astype(jnp.bfloat16)
        xr = jnp.dot(x2, w_ref[...], preferred_element_type=jnp.float32)
        xr = ((xr + b_ref[...]) * (1.0 / 3.0)).reshape(G, N, H)
        v = g_ref[...]
        glo = lax.bitcast_convert_type(v << 16, jnp.float32)
        ghi = lax.bitcast_convert_type(v & jnp.int32(-65536), jnp.float32)
        gf = jnp.concatenate([glo, ghi], axis=-1)
        comb = gf + xr
        tok = jnp.broadcast_to(t_ref[...].reshape(1, 1, H), (G, 1, H))
        o_ref[...] = jnp.concatenate([tok, comb], axis=1)

    return pl.pallas_call(
        body,
        grid=(B // G,),
        in_specs=[
            pl.BlockSpec((G, N, D), lambda g: (g, 0, 0)),
            pl.BlockSpec((G, N, H2), lambda g: (g, 0, 0)),
            pl.BlockSpec((D, H), lambda g: (0, 0)),
            pl.BlockSpec((1, H), lambda g: (0, 0)),
            pl.BlockSpec((1, H), lambda g: (0, 0)),
        ],
        out_specs=pl.BlockSpec((G, N + 1, H), lambda g: (g, 0, 0)),
        out_shape=jax.ShapeDtypeStruct((B, N + 1, H), jnp.float32),
        compiler_params=pltpu.CompilerParams(
            dimension_semantics=("parallel",),
        ),
    )


def kernel(x_long, x_real, degree, long_table, real_W, real_b,
           degree_table, graph_token):
    B, N, L = x_long.shape
    D = x_real.shape[-1]
    H = long_table.shape[1]
    H2 = H // 2
    n_nodes = B * N
    n_long = long_table.shape[0]

    def pack_bf16(tab):
        bf = tab.astype(jnp.bfloat16)
        lo = lax.bitcast_convert_type(bf[:, :H2], jnp.uint16).astype(
            jnp.uint32)
        hi = lax.bitcast_convert_type(bf[:, H2:], jnp.uint16).astype(
            jnp.uint32)
        return lax.bitcast_convert_type(lo | (hi << 16), jnp.int32)

    sc_gather, NW, PB, NB, LD = _make_sc_gather(n_nodes, L, H)
    tab = jnp.concatenate([pack_bf16(long_table), pack_bf16(degree_table)])
    idx = jnp.concatenate(
        [x_long.astype(jnp.int32).reshape(n_nodes, L),
         degree.astype(jnp.int32).reshape(n_nodes, 1) + n_long],
        axis=1).reshape(n_nodes * LD)
    gath_i32 = sc_gather(idx, tab)

    tc = _make_tc_combine(B, N, D, H, 4)
    return tc(x_real, gath_i32.reshape(B, N, H2),
              real_W.astype(jnp.bfloat16),
              real_b.reshape(1, H), graph_token.reshape(1, H))
